# baseline (speedup 1.0000x reference)
import jax
import jax.numpy as jnp
from jax import lax
from jax.experimental import pallas as pl
from jax.experimental.pallas import tpu as pltpu
from jax.experimental.pallas import tpu_sc as plsc

EMB = 1000
BATCH = 1024
HIST = 50
NC, NS = 2, 16
NW = NC * NS
NET = EMB // 8
UNITS = NET * HIST
LANES = 16


def _body(xt_hbm, wt_hbm, z_hbm, xt_v, vb_v, stg_v, sem0, sem1, sem2):
    wid = lax.axis_index("s") * NC + lax.axis_index("c")
    lo = wid * UNITS // NW
    n = (wid + 1) * UNITS // NW - lo
    pltpu.sync_copy(xt_hbm, xt_v)
    sems = (sem0, sem1, sem2)
    rows = [jnp.full((LANES,), s, dtype=jnp.int32) for s in range(8)]

    def compute_unit(i, b):
        f = lo + i
        et = f // HIST
        h = f - et * HIST

        @pl.when(jnp.logical_or(i == 0, h == 0))
        def _():
            pltpu.sync_copy(wt_hbm.at[pl.ds(et * 8, 8)], vb_v)

        for bt in range(8):
            for g in range(8):
                idx16 = xt_v[h, pl.ds(bt * 128 + g * 16, LANES)]
                for sl in range(8):
                    val = plsc.load_gather(vb_v, [rows[sl], idx16])
                    stg_v[b, bt, sl, pl.ds(g * 16, LANES)] = val
        pltpu.async_copy(stg_v.at[b], z_hbm.at[h, et], sems[b])

    def drain(b):
        pltpu.make_async_copy(stg_v.at[b], z_hbm.at[0, 0], sems[b]).wait()

    npairs = n // 2

    def pair_body(j, carry):
        for b in range(2):
            @pl.when(j >= 1)
            def _():
                drain(b)
            compute_unit(2 * j + b, b)
        return carry

    lax.fori_loop(0, npairs, pair_body, 0)

    @pl.when(npairs >= 1)
    def _():
        drain(0)
        drain(1)

    @pl.when(n % 2 == 1)
    def _():
        compute_unit(n - 1, 2)
        drain(2)


def kernel(x, W):
    xt = x.T
    wt = W.T
    mesh = plsc.VectorSubcoreMesh(core_axis_name="c", subcore_axis_name="s")
    z = pl.kernel(
        _body,
        out_type=jax.ShapeDtypeStruct((HIST, NET, 8, 8, 128), jnp.float32),
        mesh=mesh,
        scratch_types=[
            pltpu.VMEM((HIST, BATCH), jnp.int32),
            pltpu.VMEM((8, EMB), jnp.float32),
            pltpu.VMEM((3, 8, 8, 128), jnp.float32),
            pltpu.SemaphoreType.DMA,
            pltpu.SemaphoreType.DMA,
            pltpu.SemaphoreType.DMA,
        ],
        compiler_params=pltpu.CompilerParams(
            use_tc_tiling_on_sc=False, needs_layout_passes=False),
    )(xt, wt)
    return z.transpose(2, 4, 0, 1, 3).reshape(BATCH, HIST, EMB)


# speedup vs baseline: 1.7594x; 1.7594x over previous
import jax
import jax.numpy as jnp
from jax import lax
from jax.experimental import pallas as pl
from jax.experimental.pallas import tpu as pltpu
from jax.experimental.pallas import tpu_sc as plsc

EMB = 1000
BATCH = 1024
HIST = 50
NC, NS = 2, 16
NW = NC * NS
NET = EMB // 8
UNITS = NET * HIST
LANES = 16


def _body(xt_hbm, wt_hbm, z_hbm, xt_v, vb_v, stg_v, sem0, sem1, sem2):
    wid = lax.axis_index("s") * NC + lax.axis_index("c")
    lo = wid * UNITS // NW
    n = (wid + 1) * UNITS // NW - lo
    pltpu.sync_copy(xt_hbm, xt_v)
    sems = (sem0, sem1, sem2)
    rows = [jnp.full((LANES,), s, dtype=jnp.int32) for s in range(8)]

    def compute_unit(i, b):
        f = lo + i
        et = f // HIST
        h = f - et * HIST

        @pl.when(jnp.logical_or(i == 0, h == 0))
        def _():
            pltpu.sync_copy(wt_hbm.at[pl.ds(et * 8, 8)], vb_v)

        for bt in range(8):
            for g in range(0, 8, 2):
                idx_a = xt_v[h, pl.ds(bt * 128 + g * 16, LANES)]
                idx_b = xt_v[h, pl.ds(bt * 128 + (g + 1) * 16, LANES)]
                vals_a = [plsc.load_gather(vb_v, [rows[sl], idx_a])
                          for sl in range(8)]
                vals_b = [plsc.load_gather(vb_v, [rows[sl], idx_b])
                          for sl in range(8)]
                for sl in range(8):
                    stg_v[b, bt, sl, pl.ds(g * 16, LANES)] = vals_a[sl]
                    stg_v[b, bt, sl, pl.ds((g + 1) * 16, LANES)] = vals_b[sl]
        pltpu.async_copy(stg_v.at[b], z_hbm.at[h, et], sems[b])

    def drain(b):
        pltpu.make_async_copy(stg_v.at[b], z_hbm.at[0, 0], sems[b]).wait()

    npairs = n // 2

    def pair_body(j, carry):
        for b in range(2):
            @pl.when(j >= 1)
            def _():
                drain(b)
            compute_unit(2 * j + b, b)
        return carry

    lax.fori_loop(0, npairs, pair_body, 0)

    @pl.when(npairs >= 1)
    def _():
        drain(0)
        drain(1)

    @pl.when(n % 2 == 1)
    def _():
        compute_unit(n - 1, 2)
        drain(2)


def kernel(x, W):
    xt = x.T
    wt = W.T
    mesh = plsc.VectorSubcoreMesh(core_axis_name="c", subcore_axis_name="s")
    z = pl.kernel(
        _body,
        out_type=jax.ShapeDtypeStruct((HIST, NET, 8, 8, 128), jnp.float32),
        mesh=mesh,
        scratch_types=[
            pltpu.VMEM((HIST, BATCH), jnp.int32),
            pltpu.VMEM((8, EMB), jnp.float32),
            pltpu.VMEM((3, 8, 8, 128), jnp.float32),
            pltpu.SemaphoreType.DMA,
            pltpu.SemaphoreType.DMA,
            pltpu.SemaphoreType.DMA,
        ],
        compiler_params=pltpu.CompilerParams(
            use_tc_tiling_on_sc=False, needs_layout_passes=False),
    )(xt, wt)
    return z.transpose(2, 4, 0, 1, 3).reshape(BATCH, HIST, EMB)


# speedup vs baseline: 1.7704x; 1.0062x over previous
import jax
import jax.numpy as jnp
from jax import lax
from jax.experimental import pallas as pl
from jax.experimental.pallas import tpu as pltpu
from jax.experimental.pallas import tpu_sc as plsc

EMB = 1000
BATCH = 1024
HIST = 50
NC, NS = 2, 16
NW = NC * NS
NET = EMB // 8
UNITS = NET * HIST
LANES = 16


def _body(xt_hbm, wt_hbm, z_hbm, xt_v, vb_v, stg_v, sem0, sem1, sem2):
    wid = lax.axis_index("s") * NC + lax.axis_index("c")
    lo = wid * UNITS // NW
    n = (wid + 1) * UNITS // NW - lo
    pltpu.sync_copy(xt_hbm, xt_v)
    sems = (sem0, sem1, sem2)
    rows = [jnp.full((LANES,), s, dtype=jnp.int32) for s in range(8)]

    def compute_unit(i, b):
        f = lo + i
        et = f // HIST
        h = f - et * HIST

        @pl.when(jnp.logical_or(i == 0, h == 0))
        def _():
            pltpu.sync_copy(wt_hbm.at[pl.ds(et * 8, 8)], vb_v)

        def gather_batch(bt, g0):
            out = []
            for k in range(2):
                idx = xt_v[h, pl.ds(bt * 128 + (g0 + k) * 16, LANES)]
                out.append([plsc.load_gather(vb_v, [rows[sl], idx])
                            for sl in range(8)])
            return out

        def store_batch(bt, g0, vals):
            for k in range(2):
                for sl in range(8):
                    stg_v[b, bt, sl, pl.ds((g0 + k) * 16, LANES)] = vals[k][sl]

        pending = None
        for bt in range(8):
            for g0 in range(0, 8, 2):
                vals = gather_batch(bt, g0)
                if pending is not None:
                    store_batch(*pending)
                pending = (bt, g0, vals)
        store_batch(*pending)
        pltpu.async_copy(stg_v.at[b], z_hbm.at[h, et], sems[b])

    def drain(b):
        pltpu.make_async_copy(stg_v.at[b], z_hbm.at[0, 0], sems[b]).wait()

    npairs = n // 2

    def pair_body(j, carry):
        for b in range(2):
            @pl.when(j >= 1)
            def _():
                drain(b)
            compute_unit(2 * j + b, b)
        return carry

    lax.fori_loop(0, npairs, pair_body, 0)

    @pl.when(npairs >= 1)
    def _():
        drain(0)
        drain(1)

    @pl.when(n % 2 == 1)
    def _():
        compute_unit(n - 1, 2)
        drain(2)


def kernel(x, W):
    xt = x.T
    wt = W.T
    mesh = plsc.VectorSubcoreMesh(core_axis_name="c", subcore_axis_name="s")
    z = pl.kernel(
        _body,
        out_type=jax.ShapeDtypeStruct((HIST, NET, 8, 8, 128), jnp.float32),
        mesh=mesh,
        scratch_types=[
            pltpu.VMEM((HIST, BATCH), jnp.int32),
            pltpu.VMEM((8, EMB), jnp.float32),
            pltpu.VMEM((3, 8, 8, 128), jnp.float32),
            pltpu.SemaphoreType.DMA,
            pltpu.SemaphoreType.DMA,
            pltpu.SemaphoreType.DMA,
        ],
        compiler_params=pltpu.CompilerParams(
            use_tc_tiling_on_sc=False, needs_layout_passes=False),
    )(xt, wt)
    return z.transpose(2, 4, 0, 1, 3).reshape(BATCH, HIST, EMB)
